# baseline (speedup 1.0000x reference)
import functools

import jax
import jax.numpy as jnp
from jax import lax
from jax.experimental import pallas as pl
from jax.experimental.pallas import tpu as pltpu
from jax.experimental.pallas import tpu_sc as plsc

_NT = 16384
_D = 211
_NE = 100
_EP = 104
_TBLK = 2048
_NW = 32
_CHUNK = _NT // _NW
_NG = _CHUNK // 16


def _score_body(x_ref, w_ref, b_ref, o_ref):
    logits = lax.dot_general(
        w_ref[...], x_ref[...],
        dimension_numbers=(((1,), (1,)), ((), ())),
        preferred_element_type=jnp.float32,
    )
    o_ref[...] = jax.nn.sigmoid(logits + b_ref[...])


def _scores_tc(x, w_pad, b_pad):
    return pl.pallas_call(
        _score_body,
        grid=(_NW,),
        in_specs=[
            pl.BlockSpec((_TBLK, _D), lambda i: (i, 0)),
            pl.BlockSpec((_EP, _D), lambda i: (0, 0)),
            pl.BlockSpec((_EP, 1), lambda i: (0, 0)),
        ],
        out_specs=pl.BlockSpec((_EP, _TBLK), lambda i: (0, i)),
        out_shape=jax.ShapeDtypeStruct((_EP, _NT), jnp.float32),
    )(x, w_pad, b_pad)


def _router_sc(scores_b):
    mesh = plsc.VectorSubcoreMesh(core_axis_name="c", subcore_axis_name="s")

    @functools.partial(
        pl.kernel,
        mesh=mesh,
        out_type=[
            jax.ShapeDtypeStruct((_NT,), jnp.float32),
            jax.ShapeDtypeStruct((_NT,), jnp.float32),
            jax.ShapeDtypeStruct((_NT,), jnp.int32),
            jax.ShapeDtypeStruct((_NT,), jnp.int32),
        ],
        scratch_types=[
            pltpu.VMEM((_EP, _CHUNK), jnp.float32),
            pltpu.VMEM((_CHUNK,), jnp.float32),
            pltpu.VMEM((_CHUNK,), jnp.float32),
            pltpu.VMEM((_CHUNK,), jnp.int32),
            pltpu.VMEM((_CHUNK,), jnp.int32),
        ],
    )
    def k(scores_hbm, w1_hbm, w2_hbm, i1_hbm, i2_hbm,
          sc_v, w1_v, w2_v, i1_v, i2_v):
        wid = lax.axis_index("c") * 16 + lax.axis_index("s")
        base = wid * _CHUNK
        pltpu.sync_copy(scores_hbm.at[:, pl.ds(base, _CHUNK)], sc_v)

        def scan_group(off):
            m1 = jnp.full((16,), -jnp.inf, jnp.float32)
            m2 = jnp.full((16,), -jnp.inf, jnp.float32)
            i1 = jnp.zeros((16,), jnp.int32)
            i2 = jnp.zeros((16,), jnp.int32)
            for e in range(_NE):
                col = jnp.full((16,), e, jnp.int32)
                v = sc_v[e, pl.ds(off, 16)]
                gt1 = v > m1
                gt2 = v > m2
                m2 = jnp.where(gt1, m1, jnp.where(gt2, v, m2))
                i2 = jnp.where(gt1, i1, jnp.where(gt2, col, i2))
                m1 = jnp.where(gt1, v, m1)
                i1 = jnp.where(gt1, col, i1)
            s = m1 + m2
            w1_v[pl.ds(off, 16)] = m1 / s
            w2_v[pl.ds(off, 16)] = m2 / s
            i1_v[pl.ds(off, 16)] = i1
            i2_v[pl.ds(off, 16)] = i2

        def body(g, carry):
            scan_group(g * 32)
            scan_group(g * 32 + 16)
            return carry

        lax.fori_loop(0, _NG // 2, body, 0)
        pltpu.sync_copy(w1_v, w1_hbm.at[pl.ds(base, _CHUNK)])
        pltpu.sync_copy(w2_v, w2_hbm.at[pl.ds(base, _CHUNK)])
        pltpu.sync_copy(i1_v, i1_hbm.at[pl.ds(base, _CHUNK)])
        pltpu.sync_copy(i2_v, i2_hbm.at[pl.ds(base, _CHUNK)])

    return k(scores_b)


def kernel(x, W, b):
    w_pad = jnp.pad(W, ((0, _EP - _NE), (0, 0)))
    b_pad = jnp.pad(b, (0, _EP - _NE)).reshape(_EP, 1)
    scores_t = _scores_tc(x, w_pad, b_pad)
    w1, w2, i1, i2 = _router_sc(scores_t)
    weights = jnp.stack([w1, w2], axis=1)
    indices = jnp.stack([i1, i2], axis=1)
    return weights.astype(x.dtype), indices


# speedup vs baseline: 1.5443x; 1.5443x over previous
import functools

import jax
import jax.numpy as jnp
from jax import lax
from jax.experimental import pallas as pl
from jax.experimental.pallas import tpu as pltpu
from jax.experimental.pallas import tpu_sc as plsc

_NT = 16384
_D = 211
_NE = 100
_EP = 104
_TBLK = 2048
_NW = 32
_CHUNK = _NT // _NW
_NG = _CHUNK // 16


def _score_body(x_ref, w_ref, b_ref, o_ref):
    logits = lax.dot_general(
        w_ref[...], x_ref[...],
        dimension_numbers=(((1,), (1,)), ((), ())),
        preferred_element_type=jnp.float32,
    )
    o_ref[...] = jax.nn.sigmoid(logits + b_ref[...])


def _scores_tc(x, w_pad, b_pad):
    return pl.pallas_call(
        _score_body,
        grid=(_NT // _TBLK,),
        in_specs=[
            pl.BlockSpec((_TBLK, _D), lambda i: (i, 0)),
            pl.BlockSpec((_EP, _D), lambda i: (0, 0)),
            pl.BlockSpec((_EP, 1), lambda i: (0, 0)),
        ],
        out_specs=pl.BlockSpec((_EP, _TBLK), lambda i: (0, i)),
        out_shape=jax.ShapeDtypeStruct((_EP, _NT), jnp.float32),
    )(x, w_pad, b_pad)


def _router_sc(scores_b):
    mesh = plsc.VectorSubcoreMesh(core_axis_name="c", subcore_axis_name="s")

    @functools.partial(
        pl.kernel,
        mesh=mesh,
        out_type=[
            jax.ShapeDtypeStruct((_NT,), jnp.float32),
            jax.ShapeDtypeStruct((_NT,), jnp.float32),
            jax.ShapeDtypeStruct((_NT,), jnp.int32),
            jax.ShapeDtypeStruct((_NT,), jnp.int32),
        ],
        scratch_types=[
            pltpu.VMEM((_EP, _CHUNK), jnp.float32),
            pltpu.VMEM((_CHUNK,), jnp.float32),
            pltpu.VMEM((_CHUNK,), jnp.float32),
            pltpu.VMEM((_CHUNK,), jnp.int32),
            pltpu.VMEM((_CHUNK,), jnp.int32),
        ],
    )
    def k(scores_hbm, w1_hbm, w2_hbm, i1_hbm, i2_hbm,
          sc_v, w1_v, w2_v, i1_v, i2_v):
        wid = lax.axis_index("c") * 16 + lax.axis_index("s")
        base = wid * _CHUNK
        pltpu.sync_copy(scores_hbm.at[:, pl.ds(base, _CHUNK)], sc_v)

        def scan_group(off):
            m1 = jnp.full((16,), -jnp.inf, jnp.float32)
            m2 = jnp.full((16,), -jnp.inf, jnp.float32)
            i1 = jnp.zeros((16,), jnp.int32)
            i2 = jnp.zeros((16,), jnp.int32)
            for e in range(_NE):
                col = jnp.full((16,), e, jnp.int32)
                v = sc_v[e, pl.ds(off, 16)]
                gt1 = v > m1
                gt2 = v > m2
                m2 = jnp.where(gt1, m1, jnp.where(gt2, v, m2))
                i2 = jnp.where(gt1, i1, jnp.where(gt2, col, i2))
                m1 = jnp.where(gt1, v, m1)
                i1 = jnp.where(gt1, col, i1)
            s = m1 + m2
            w1_v[pl.ds(off, 16)] = m1 / s
            w2_v[pl.ds(off, 16)] = m2 / s
            i1_v[pl.ds(off, 16)] = i1
            i2_v[pl.ds(off, 16)] = i2

        def body(g, carry):
            scan_group(g * 32)
            scan_group(g * 32 + 16)
            return carry

        lax.fori_loop(0, _NG // 2, body, 0)
        pltpu.sync_copy(w1_v, w1_hbm.at[pl.ds(base, _CHUNK)])
        pltpu.sync_copy(w2_v, w2_hbm.at[pl.ds(base, _CHUNK)])
        pltpu.sync_copy(i1_v, i1_hbm.at[pl.ds(base, _CHUNK)])
        pltpu.sync_copy(i2_v, i2_hbm.at[pl.ds(base, _CHUNK)])

    return k(scores_b)


def kernel(x, W, b):
    w_pad = jnp.pad(W, ((0, _EP - _NE), (0, 0)))
    b_pad = jnp.pad(b, (0, _EP - _NE)).reshape(_EP, 1)
    scores_t = _scores_tc(x, w_pad, b_pad)
    w1, w2, i1, i2 = _router_sc(scores_t)
    weights = jnp.stack([w1, w2], axis=1)
    indices = jnp.stack([i1, i2], axis=1)
    return weights.astype(x.dtype), indices


# speedup vs baseline: 1.5573x; 1.0085x over previous
import functools

import jax
import jax.numpy as jnp
from jax import lax
from jax.experimental import pallas as pl
from jax.experimental.pallas import tpu as pltpu
from jax.experimental.pallas import tpu_sc as plsc

_NT = 16384
_D = 211
_NE = 100
_EP = 104
_TBLK = 2048
_NW = 32
_CHUNK = _NT // _NW
_NG = _CHUNK // 16


_SUB = _TBLK // _CHUNK


def _score_body(x_ref, w_ref, b_ref, o_ref):
    w = jnp.concatenate(
        [w_ref[...], jnp.zeros((_EP - _NE, _D), jnp.float32)], axis=0)
    b = jnp.concatenate(
        [b_ref[...], jnp.zeros((_EP - _NE, 1), jnp.float32)], axis=0)
    for j in range(_SUB):
        logits = lax.dot_general(
            w, x_ref[pl.ds(j * _CHUNK, _CHUNK), :],
            dimension_numbers=(((1,), (1,)), ((), ())),
            preferred_element_type=jnp.float32,
        )
        o_ref[j] = jax.nn.sigmoid(logits + b)


def _scores_tc(x, w_raw, b_raw):
    return pl.pallas_call(
        _score_body,
        grid=(_NT // _TBLK,),
        in_specs=[
            pl.BlockSpec((_TBLK, _D), lambda i: (i, 0)),
            pl.BlockSpec((_NE, _D), lambda i: (0, 0)),
            pl.BlockSpec((_NE, 1), lambda i: (0, 0)),
        ],
        out_specs=pl.BlockSpec((_SUB, _EP, _CHUNK), lambda i: (i, 0, 0)),
        out_shape=jax.ShapeDtypeStruct((_NW, _EP, _CHUNK), jnp.float32),
    )(x, w_raw, b_raw)


def _router_sc(scores_b):
    mesh = plsc.VectorSubcoreMesh(core_axis_name="c", subcore_axis_name="s")

    @functools.partial(
        pl.kernel,
        mesh=mesh,
        out_type=[
            jax.ShapeDtypeStruct((_NT,), jnp.float32),
            jax.ShapeDtypeStruct((_NT,), jnp.float32),
            jax.ShapeDtypeStruct((_NT,), jnp.int32),
            jax.ShapeDtypeStruct((_NT,), jnp.int32),
        ],
        scratch_types=[
            pltpu.VMEM((_EP, _CHUNK), jnp.float32),
            pltpu.VMEM((_CHUNK,), jnp.float32),
            pltpu.VMEM((_CHUNK,), jnp.float32),
            pltpu.VMEM((_CHUNK,), jnp.int32),
            pltpu.VMEM((_CHUNK,), jnp.int32),
        ],
    )
    def k(scores_hbm, w1_hbm, w2_hbm, i1_hbm, i2_hbm,
          sc_v, w1_v, w2_v, i1_v, i2_v):
        wid = lax.axis_index("c") * 16 + lax.axis_index("s")
        base = wid * _CHUNK
        pltpu.sync_copy(scores_hbm.at[wid], sc_v)

        def scan_group(off):
            m1 = jnp.full((16,), -jnp.inf, jnp.float32)
            m2 = jnp.full((16,), -jnp.inf, jnp.float32)
            i1 = jnp.zeros((16,), jnp.int32)
            i2 = jnp.zeros((16,), jnp.int32)
            for e in range(_NE):
                col = jnp.full((16,), e, jnp.int32)
                v = sc_v[e, pl.ds(off, 16)]
                gt1 = v > m1
                gt2 = v > m2
                m2 = jnp.where(gt1, m1, jnp.where(gt2, v, m2))
                i2 = jnp.where(gt1, i1, jnp.where(gt2, col, i2))
                m1 = jnp.where(gt1, v, m1)
                i1 = jnp.where(gt1, col, i1)
            s = m1 + m2
            w1_v[pl.ds(off, 16)] = m1 / s
            w2_v[pl.ds(off, 16)] = m2 / s
            i1_v[pl.ds(off, 16)] = i1
            i2_v[pl.ds(off, 16)] = i2

        def body(g, carry):
            scan_group(g * 32)
            scan_group(g * 32 + 16)
            return carry

        lax.fori_loop(0, _NG // 2, body, 0)
        pltpu.sync_copy(w1_v, w1_hbm.at[pl.ds(base, _CHUNK)])
        pltpu.sync_copy(w2_v, w2_hbm.at[pl.ds(base, _CHUNK)])
        pltpu.sync_copy(i1_v, i1_hbm.at[pl.ds(base, _CHUNK)])
        pltpu.sync_copy(i2_v, i2_hbm.at[pl.ds(base, _CHUNK)])

    return k(scores_b)


def kernel(x, W, b):
    scores_t = _scores_tc(x, W, b.reshape(_NE, 1))
    w1, w2, i1, i2 = _router_sc(scores_t)
    weights = jnp.stack([w1, w2], axis=1)
    indices = jnp.stack([i1, i2], axis=1)
    return weights.astype(x.dtype), indices


# speedup vs baseline: 1.5876x; 1.0194x over previous
import functools

import jax
import jax.numpy as jnp
from jax import lax
from jax.experimental import pallas as pl
from jax.experimental.pallas import tpu as pltpu
from jax.experimental.pallas import tpu_sc as plsc

_NT = 16384
_D = 211
_NE = 100
_EP = 104
_TBLK = 2048
_NW = 32
_CHUNK = _NT // _NW
_NG = _CHUNK // 16


_SUB = _TBLK // _CHUNK


def _score_body(x_ref, w_ref, b_ref, o_ref):
    w = jnp.concatenate(
        [w_ref[...], jnp.zeros((_EP - _NE, _D), jnp.float32)], axis=0)
    b = jnp.concatenate(
        [b_ref[...], jnp.zeros((_EP - _NE, 1), jnp.float32)], axis=0)
    for j in range(_SUB):
        logits = lax.dot_general(
            w, x_ref[pl.ds(j * _CHUNK, _CHUNK), :],
            dimension_numbers=(((1,), (1,)), ((), ())),
            preferred_element_type=jnp.float32,
        )
        o_ref[j] = jax.nn.sigmoid(logits + b)


def _scores_tc(x, w_raw, b_raw):
    return pl.pallas_call(
        _score_body,
        grid=(_NT // _TBLK,),
        in_specs=[
            pl.BlockSpec((_TBLK, _D), lambda i: (i, 0)),
            pl.BlockSpec((_NE, _D), lambda i: (0, 0)),
            pl.BlockSpec((_NE, 1), lambda i: (0, 0)),
        ],
        out_specs=pl.BlockSpec((_SUB, _EP, _CHUNK), lambda i: (i, 0, 0)),
        out_shape=jax.ShapeDtypeStruct((_NW, _EP, _CHUNK), jnp.float32),
    )(x, w_raw, b_raw)


def _router_sc(scores_b):
    mesh = plsc.VectorSubcoreMesh(core_axis_name="c", subcore_axis_name="s")

    @functools.partial(
        pl.kernel,
        mesh=mesh,
        out_type=[
            jax.ShapeDtypeStruct((_NT,), jnp.float32),
            jax.ShapeDtypeStruct((_NT,), jnp.float32),
            jax.ShapeDtypeStruct((_NT,), jnp.int32),
            jax.ShapeDtypeStruct((_NT,), jnp.int32),
        ],
        scratch_types=[
            pltpu.VMEM((_EP, _CHUNK), jnp.float32),
            pltpu.VMEM((_CHUNK,), jnp.float32),
            pltpu.VMEM((_CHUNK,), jnp.float32),
            pltpu.VMEM((_CHUNK,), jnp.int32),
            pltpu.VMEM((_CHUNK,), jnp.int32),
        ],
    )
    def k(scores_hbm, w1_hbm, w2_hbm, i1_hbm, i2_hbm,
          sc_v, w1_v, w2_v, i1_v, i2_v):
        wid = lax.axis_index("c") * 16 + lax.axis_index("s")
        base = wid * _CHUNK
        pltpu.sync_copy(scores_hbm.at[wid], sc_v)

        def scan_group(off):
            def estep(eo, carry):
                m1, m2, i1, i2 = carry
                for k in range(4):
                    e = eo * 4 + k
                    col = jnp.full((16,), 1, jnp.int32) * e
                    v = sc_v[pl.ds(e, 1), pl.ds(off, 16)].reshape(16)
                    gt1 = v > m1
                    gt2 = v > m2
                    m2 = jnp.where(gt1, m1, jnp.where(gt2, v, m2))
                    i2 = jnp.where(gt1, i1, jnp.where(gt2, col, i2))
                    m1 = jnp.where(gt1, v, m1)
                    i1 = jnp.where(gt1, col, i1)
                return m1, m2, i1, i2

            init = (jnp.full((16,), -jnp.inf, jnp.float32),
                    jnp.full((16,), -jnp.inf, jnp.float32),
                    jnp.zeros((16,), jnp.int32),
                    jnp.zeros((16,), jnp.int32))
            m1, m2, i1, i2 = lax.fori_loop(0, _NE // 4, estep, init)
            s = m1 + m2
            w1_v[pl.ds(off, 16)] = m1 / s
            w2_v[pl.ds(off, 16)] = m2 / s
            i1_v[pl.ds(off, 16)] = i1
            i2_v[pl.ds(off, 16)] = i2

        def body(g, carry):
            scan_group(g * 32)
            scan_group(g * 32 + 16)
            return carry

        lax.fori_loop(0, _NG // 2, body, 0)
        pltpu.sync_copy(w1_v, w1_hbm.at[pl.ds(base, _CHUNK)])
        pltpu.sync_copy(w2_v, w2_hbm.at[pl.ds(base, _CHUNK)])
        pltpu.sync_copy(i1_v, i1_hbm.at[pl.ds(base, _CHUNK)])
        pltpu.sync_copy(i2_v, i2_hbm.at[pl.ds(base, _CHUNK)])

    return k(scores_b)


def kernel(x, W, b):
    scores_t = _scores_tc(x, W, b.reshape(_NE, 1))
    w1, w2, i1, i2 = _router_sc(scores_t)
    weights = jnp.stack([w1, w2], axis=1)
    indices = jnp.stack([i1, i2], axis=1)
    return weights.astype(x.dtype), indices


# speedup vs baseline: 2.1844x; 1.3759x over previous
import functools

import jax
import jax.numpy as jnp
from jax import lax
from jax.experimental import pallas as pl
from jax.experimental.pallas import tpu as pltpu
from jax.experimental.pallas import tpu_sc as plsc

_NT = 16384
_D = 211
_NE = 100
_EP = 104
_TBLK = 2048
_NW = 32
_CHUNK = _NT // _NW
_NG = _CHUNK // 16


_SUB = _TBLK // _CHUNK


def _score_body(xt_ref, w_ref, b_ref, o_ref):
    w = jnp.concatenate(
        [w_ref[...], jnp.zeros((_EP - _NE, _D), jnp.float32)], axis=0)
    b = jnp.concatenate(
        [b_ref[...], jnp.zeros((_EP - _NE, 1), jnp.float32)], axis=0)
    for j in range(_SUB):
        logits = lax.dot_general(
            w, xt_ref[:, pl.ds(j * _CHUNK, _CHUNK)],
            dimension_numbers=(((1,), (0,)), ((), ())),
            preferred_element_type=jnp.float32,
        )
        o_ref[pl.ds(j, 1)] = jax.nn.sigmoid(logits + b)[None]


def _scores_tc(xt, w_raw, b_raw):
    return pl.pallas_call(
        _score_body,
        grid=(_NT // _TBLK,),
        in_specs=[
            pl.BlockSpec((_D, _TBLK), lambda i: (0, i)),
            pl.BlockSpec((_NE, _D), lambda i: (0, 0)),
            pl.BlockSpec((_NE, 1), lambda i: (0, 0)),
        ],
        out_specs=pl.BlockSpec((_SUB, _EP, _CHUNK), lambda i: (i, 0, 0)),
        out_shape=jax.ShapeDtypeStruct((_NW, _EP, _CHUNK), jnp.float32),
    )(xt, w_raw, b_raw)


def _router_sc(scores_b):
    mesh = plsc.VectorSubcoreMesh(core_axis_name="c", subcore_axis_name="s")

    @functools.partial(
        pl.kernel,
        mesh=mesh,
        out_type=[
            jax.ShapeDtypeStruct((_NT,), jnp.float32),
            jax.ShapeDtypeStruct((_NT,), jnp.float32),
            jax.ShapeDtypeStruct((_NT,), jnp.int32),
            jax.ShapeDtypeStruct((_NT,), jnp.int32),
        ],
        scratch_types=[
            pltpu.VMEM((_EP, _CHUNK), jnp.float32),
            pltpu.VMEM((_CHUNK,), jnp.float32),
            pltpu.VMEM((_CHUNK,), jnp.float32),
            pltpu.VMEM((_CHUNK,), jnp.int32),
            pltpu.VMEM((_CHUNK,), jnp.int32),
        ],
    )
    def k(scores_hbm, w1_hbm, w2_hbm, i1_hbm, i2_hbm,
          sc_v, w1_v, w2_v, i1_v, i2_v):
        wid = lax.axis_index("c") * 16 + lax.axis_index("s")
        base = wid * _CHUNK
        pltpu.sync_copy(scores_hbm.at[wid], sc_v)

        def scan_group(off):
            def estep(eo, carry):
                m1, m2, i1, i2 = carry
                for k in range(4):
                    e = eo * 4 + k
                    col = jnp.full((16,), 1, jnp.int32) * e
                    v = sc_v[pl.ds(e, 1), pl.ds(off, 16)].reshape(16)
                    gt1 = v > m1
                    gt2 = v > m2
                    m2 = jnp.where(gt1, m1, jnp.where(gt2, v, m2))
                    i2 = jnp.where(gt1, i1, jnp.where(gt2, col, i2))
                    m1 = jnp.where(gt1, v, m1)
                    i1 = jnp.where(gt1, col, i1)
                return m1, m2, i1, i2

            init = (jnp.full((16,), -jnp.inf, jnp.float32),
                    jnp.full((16,), -jnp.inf, jnp.float32),
                    jnp.zeros((16,), jnp.int32),
                    jnp.zeros((16,), jnp.int32))
            m1, m2, i1, i2 = lax.fori_loop(0, _NE // 4, estep, init)
            s = m1 + m2
            w1_v[pl.ds(off, 16)] = m1 / s
            w2_v[pl.ds(off, 16)] = m2 / s
            i1_v[pl.ds(off, 16)] = i1
            i2_v[pl.ds(off, 16)] = i2

        def body(g, carry):
            scan_group(g * 32)
            scan_group(g * 32 + 16)
            return carry

        lax.fori_loop(0, _NG // 2, body, 0)
        pltpu.sync_copy(w1_v, w1_hbm.at[pl.ds(base, _CHUNK)])
        pltpu.sync_copy(w2_v, w2_hbm.at[pl.ds(base, _CHUNK)])
        pltpu.sync_copy(i1_v, i1_hbm.at[pl.ds(base, _CHUNK)])
        pltpu.sync_copy(i2_v, i2_hbm.at[pl.ds(base, _CHUNK)])

    return k(scores_b)


def kernel(x, W, b):
    scores_t = _scores_tc(x.T, W, b.reshape(_NE, 1))
    w1, w2, i1, i2 = _router_sc(scores_t)
    weights = jnp.stack([w1, w2], axis=1)
    indices = jnp.stack([i1, i2], axis=1)
    return weights.astype(x.dtype), indices


# speedup vs baseline: 2.3056x; 1.0555x over previous
import functools

import jax
import jax.numpy as jnp
from jax import lax
from jax.experimental import pallas as pl
from jax.experimental.pallas import tpu as pltpu
from jax.experimental.pallas import tpu_sc as plsc

_NT = 16384
_D = 211
_NE = 100
_EP = 104
_TBLK = 2048
_NW = 32
_CHUNK = _NT // _NW
_NG = _CHUNK // 16


_SUB = _TBLK // _CHUNK


def _score_body(xt_ref, w_ref, b_ref, o_ref):
    w = jnp.concatenate(
        [w_ref[...], jnp.zeros((_EP - _NE, _D), jnp.float32)], axis=0)
    brow = jnp.concatenate(
        [b_ref[...], jnp.zeros((1, _EP - _NE), jnp.float32)], axis=1)
    b = lax.transpose(brow, (1, 0))
    for j in range(_NW):
        logits = lax.dot_general(
            w, xt_ref[:, pl.ds(j * _CHUNK, _CHUNK)],
            dimension_numbers=(((1,), (0,)), ((), ())),
            preferred_element_type=jnp.float32,
        )
        o_ref[pl.ds(j, 1)] = jax.nn.sigmoid(logits + b)[None]


def _scores_tc(xt, w_raw, b_raw):
    return pl.pallas_call(
        _score_body,
        in_specs=[
            pl.BlockSpec(memory_space=pltpu.VMEM),
            pl.BlockSpec(memory_space=pltpu.VMEM),
            pl.BlockSpec(memory_space=pltpu.VMEM),
        ],
        out_specs=pl.BlockSpec(memory_space=pltpu.VMEM),
        out_shape=jax.ShapeDtypeStruct((_NW, _EP, _CHUNK), jnp.float32),
    )(xt, w_raw, b_raw)


def _router_sc(scores_b):
    mesh = plsc.VectorSubcoreMesh(core_axis_name="c", subcore_axis_name="s")

    @functools.partial(
        pl.kernel,
        mesh=mesh,
        out_type=[
            jax.ShapeDtypeStruct((_NT,), jnp.float32),
            jax.ShapeDtypeStruct((_NT,), jnp.float32),
            jax.ShapeDtypeStruct((_NT,), jnp.int32),
            jax.ShapeDtypeStruct((_NT,), jnp.int32),
        ],
        scratch_types=[
            pltpu.VMEM((_EP, _CHUNK), jnp.float32),
            pltpu.VMEM((_CHUNK,), jnp.float32),
            pltpu.VMEM((_CHUNK,), jnp.float32),
            pltpu.VMEM((_CHUNK,), jnp.int32),
            pltpu.VMEM((_CHUNK,), jnp.int32),
        ],
    )
    def k(scores_hbm, w1_hbm, w2_hbm, i1_hbm, i2_hbm,
          sc_v, w1_v, w2_v, i1_v, i2_v):
        wid = lax.axis_index("c") * 16 + lax.axis_index("s")
        base = wid * _CHUNK
        pltpu.sync_copy(scores_hbm.at[wid], sc_v)

        def scan_group(off):
            def estep(eo, carry):
                m1, m2, i1, i2 = carry
                for k in range(4):
                    e = eo * 4 + k
                    col = jnp.full((16,), 1, jnp.int32) * e
                    v = sc_v[pl.ds(e, 1), pl.ds(off, 16)].reshape(16)
                    gt1 = v > m1
                    gt2 = v > m2
                    m2 = jnp.where(gt1, m1, jnp.where(gt2, v, m2))
                    i2 = jnp.where(gt1, i1, jnp.where(gt2, col, i2))
                    m1 = jnp.where(gt1, v, m1)
                    i1 = jnp.where(gt1, col, i1)
                return m1, m2, i1, i2

            init = (jnp.full((16,), -jnp.inf, jnp.float32),
                    jnp.full((16,), -jnp.inf, jnp.float32),
                    jnp.zeros((16,), jnp.int32),
                    jnp.zeros((16,), jnp.int32))
            m1, m2, i1, i2 = lax.fori_loop(0, _NE // 4, estep, init)
            s = m1 + m2
            w1_v[pl.ds(off, 16)] = m1 / s
            w2_v[pl.ds(off, 16)] = m2 / s
            i1_v[pl.ds(off, 16)] = i1
            i2_v[pl.ds(off, 16)] = i2

        def body(g, carry):
            scan_group(g * 32)
            scan_group(g * 32 + 16)
            return carry

        lax.fori_loop(0, _NG // 2, body, 0)
        pltpu.sync_copy(w1_v, w1_hbm.at[pl.ds(base, _CHUNK)])
        pltpu.sync_copy(w2_v, w2_hbm.at[pl.ds(base, _CHUNK)])
        pltpu.sync_copy(i1_v, i1_hbm.at[pl.ds(base, _CHUNK)])
        pltpu.sync_copy(i2_v, i2_hbm.at[pl.ds(base, _CHUNK)])

    return k(scores_b)


def kernel(x, W, b):
    scores_t = _scores_tc(x.T, W, b.reshape(1, _NE))
    w1, w2, i1, i2 = _router_sc(scores_t)
    weights = jnp.stack([w1, w2], axis=1)
    indices = jnp.stack([i1, i2], axis=1)
    return weights.astype(x.dtype), indices


# speedup vs baseline: 2.3318x; 1.0114x over previous
import functools

import jax
import jax.numpy as jnp
from jax import lax
from jax.experimental import pallas as pl
from jax.experimental.pallas import tpu as pltpu
from jax.experimental.pallas import tpu_sc as plsc

_NT = 16384
_D = 211
_NE = 100
_EP = 104
_TBLK = 2048
_NW = 32
_CHUNK = _NT // _NW
_NG = _CHUNK // 16


_SUB = _TBLK // _CHUNK


def _score_body(xt_ref, w_ref, b_ref, o_ref):
    w = jnp.concatenate(
        [w_ref[...], jnp.zeros((_EP - _NE, _D), jnp.float32)], axis=0)
    brow = jnp.concatenate(
        [b_ref[...], jnp.zeros((1, _EP - _NE), jnp.float32)], axis=1)
    b = lax.transpose(brow, (1, 0))
    for j in range(_NW):
        logits = lax.dot_general(
            w, xt_ref[:, pl.ds(j * _CHUNK, _CHUNK)],
            dimension_numbers=(((1,), (0,)), ((), ())),
            preferred_element_type=jnp.float32,
        )
        o_ref[pl.ds(j, 1)] = jax.nn.sigmoid(logits + b)[None]


def _scores_tc(xt, w_raw, b_raw):
    return pl.pallas_call(
        _score_body,
        in_specs=[
            pl.BlockSpec(memory_space=pltpu.VMEM),
            pl.BlockSpec(memory_space=pltpu.VMEM),
            pl.BlockSpec(memory_space=pltpu.VMEM),
        ],
        out_specs=pl.BlockSpec(memory_space=pltpu.VMEM),
        out_shape=jax.ShapeDtypeStruct((_NW, _EP, _CHUNK), jnp.float32),
    )(xt, w_raw, b_raw)


def _router_sc(scores_b):
    mesh = plsc.VectorSubcoreMesh(core_axis_name="c", subcore_axis_name="s")

    @functools.partial(
        pl.kernel,
        mesh=mesh,
        out_type=[
            jax.ShapeDtypeStruct((_NT,), jnp.float32),
            jax.ShapeDtypeStruct((_NT,), jnp.float32),
            jax.ShapeDtypeStruct((_NT,), jnp.int32),
            jax.ShapeDtypeStruct((_NT,), jnp.int32),
        ],
        scratch_types=[
            pltpu.VMEM((_EP, _CHUNK), jnp.float32),
            pltpu.VMEM((_CHUNK,), jnp.float32),
            pltpu.VMEM((_CHUNK,), jnp.float32),
            pltpu.VMEM((_CHUNK,), jnp.int32),
            pltpu.VMEM((_CHUNK,), jnp.int32),
        ],
    )
    def k(scores_hbm, w1_hbm, w2_hbm, i1_hbm, i2_hbm,
          sc_v, w1_v, w2_v, i1_v, i2_v):
        wid = lax.axis_index("c") * 16 + lax.axis_index("s")
        base = wid * _CHUNK
        pltpu.sync_copy(scores_hbm.at[wid], sc_v)

        def body(g, carry):
            offa = g * 32
            offb = offa + 16

            def estep(eo, st):
                for k in range(4):
                    e = eo * 4 + k
                    col = jnp.full((16,), 1, jnp.int32) * e
                    nst = []
                    for off, (m1, m2, i1, i2) in zip((offa, offb), st):
                        v = sc_v[pl.ds(e, 1), pl.ds(off, 16)].reshape(16)
                        gt1 = v > m1
                        gt2 = v > m2
                        m2 = jnp.where(gt1, m1, jnp.where(gt2, v, m2))
                        i2 = jnp.where(gt1, i1, jnp.where(gt2, col, i2))
                        m1 = jnp.where(gt1, v, m1)
                        i1 = jnp.where(gt1, col, i1)
                        nst.append((m1, m2, i1, i2))
                    st = tuple(nst)
                return st

            init1 = (jnp.full((16,), -jnp.inf, jnp.float32),
                     jnp.full((16,), -jnp.inf, jnp.float32),
                     jnp.zeros((16,), jnp.int32),
                     jnp.zeros((16,), jnp.int32))
            sta, stb = lax.fori_loop(0, _NE // 4, estep, (init1, init1))
            for off, (m1, m2, i1, i2) in zip((offa, offb), (sta, stb)):
                s = m1 + m2
                w1_v[pl.ds(off, 16)] = m1 / s
                w2_v[pl.ds(off, 16)] = m2 / s
                i1_v[pl.ds(off, 16)] = i1
                i2_v[pl.ds(off, 16)] = i2
            return carry

        lax.fori_loop(0, _NG // 2, body, 0)
        pltpu.sync_copy(w1_v, w1_hbm.at[pl.ds(base, _CHUNK)])
        pltpu.sync_copy(w2_v, w2_hbm.at[pl.ds(base, _CHUNK)])
        pltpu.sync_copy(i1_v, i1_hbm.at[pl.ds(base, _CHUNK)])
        pltpu.sync_copy(i2_v, i2_hbm.at[pl.ds(base, _CHUNK)])

    return k(scores_b)


def kernel(x, W, b):
    scores_t = _scores_tc(x.T, W, b.reshape(1, _NE))
    w1, w2, i1, i2 = _router_sc(scores_t)
    weights = jnp.stack([w1, w2], axis=1)
    indices = jnp.stack([i1, i2], axis=1)
    return weights.astype(x.dtype), indices


# speedup vs baseline: 2.3618x; 1.0129x over previous
import functools

import jax
import jax.numpy as jnp
from jax import lax
from jax.experimental import pallas as pl
from jax.experimental.pallas import tpu as pltpu
from jax.experimental.pallas import tpu_sc as plsc

_NT = 16384
_D = 211
_NE = 100
_EP = 104
_TBLK = 2048
_NW = 32
_CHUNK = _NT // _NW
_NG = _CHUNK // 16


_SUB = _TBLK // _CHUNK


def _score_body(xt_ref, w_ref, b_ref, o_ref):
    w = jnp.concatenate(
        [w_ref[...], jnp.zeros((_EP - _NE, _D), jnp.float32)], axis=0)
    brow = jnp.concatenate(
        [b_ref[...], jnp.zeros((1, _EP - _NE), jnp.float32)], axis=1)
    b = lax.transpose(brow, (1, 0))
    for j in range(_NW):
        logits = lax.dot_general(
            w, xt_ref[:, pl.ds(j * _CHUNK, _CHUNK)],
            dimension_numbers=(((1,), (0,)), ((), ())),
            preferred_element_type=jnp.float32,
        )
        o_ref[pl.ds(j, 1)] = jax.nn.sigmoid(logits + b)[None]


def _scores_tc(xt, w_raw, b_raw):
    return pl.pallas_call(
        _score_body,
        in_specs=[
            pl.BlockSpec(memory_space=pltpu.VMEM),
            pl.BlockSpec(memory_space=pltpu.VMEM),
            pl.BlockSpec(memory_space=pltpu.VMEM),
        ],
        out_specs=pl.BlockSpec(memory_space=pltpu.VMEM),
        out_shape=jax.ShapeDtypeStruct((_NW, _EP, _CHUNK), jnp.float32),
    )(xt, w_raw, b_raw)


def _router_sc(scores_b):
    mesh = plsc.VectorSubcoreMesh(core_axis_name="c", subcore_axis_name="s")

    @functools.partial(
        pl.kernel,
        mesh=mesh,
        out_type=[
            jax.ShapeDtypeStruct((_NT,), jnp.float32),
            jax.ShapeDtypeStruct((_NT,), jnp.float32),
            jax.ShapeDtypeStruct((_NT,), jnp.int32),
            jax.ShapeDtypeStruct((_NT,), jnp.int32),
        ],
        scratch_types=[
            pltpu.VMEM((_EP, _CHUNK), jnp.float32),
            pltpu.VMEM((_CHUNK,), jnp.float32),
            pltpu.VMEM((_CHUNK,), jnp.float32),
            pltpu.VMEM((_CHUNK,), jnp.int32),
            pltpu.VMEM((_CHUNK,), jnp.int32),
        ],
    )
    def k(scores_hbm, w1_hbm, w2_hbm, i1_hbm, i2_hbm,
          sc_v, w1_v, w2_v, i1_v, i2_v):
        wid = lax.axis_index("c") * 16 + lax.axis_index("s")
        base = wid * _CHUNK
        pltpu.sync_copy(scores_hbm.at[wid], sc_v)

        def body(g, carry):
            offs = tuple(g * 64 + 16 * t for t in range(4))

            def estep(eo, st):
                for k in range(4):
                    e = eo * 4 + k
                    col = jnp.full((16,), 1, jnp.int32) * e
                    nst = []
                    for off, (m1, m2, i1, i2) in zip(offs, st):
                        v = sc_v[pl.ds(e, 1), pl.ds(off, 16)].reshape(16)
                        gt1 = v > m1
                        gt2 = v > m2
                        m2 = jnp.where(gt1, m1, jnp.where(gt2, v, m2))
                        i2 = jnp.where(gt1, i1, jnp.where(gt2, col, i2))
                        m1 = jnp.where(gt1, v, m1)
                        i1 = jnp.where(gt1, col, i1)
                        nst.append((m1, m2, i1, i2))
                    st = tuple(nst)
                return st

            init1 = (jnp.full((16,), -jnp.inf, jnp.float32),
                     jnp.full((16,), -jnp.inf, jnp.float32),
                     jnp.zeros((16,), jnp.int32),
                     jnp.zeros((16,), jnp.int32))
            sts = lax.fori_loop(0, _NE // 4, estep, (init1,) * 4)
            for off, (m1, m2, i1, i2) in zip(offs, sts):
                s = m1 + m2
                w1_v[pl.ds(off, 16)] = m1 / s
                w2_v[pl.ds(off, 16)] = m2 / s
                i1_v[pl.ds(off, 16)] = i1
                i2_v[pl.ds(off, 16)] = i2
            return carry

        lax.fori_loop(0, _NG // 4, body, 0)
        pltpu.sync_copy(w1_v, w1_hbm.at[pl.ds(base, _CHUNK)])
        pltpu.sync_copy(w2_v, w2_hbm.at[pl.ds(base, _CHUNK)])
        pltpu.sync_copy(i1_v, i1_hbm.at[pl.ds(base, _CHUNK)])
        pltpu.sync_copy(i2_v, i2_hbm.at[pl.ds(base, _CHUNK)])

    return k(scores_b)


def kernel(x, W, b):
    scores_t = _scores_tc(x.T, W, b.reshape(1, _NE))
    w1, w2, i1, i2 = _router_sc(scores_t)
    weights = jnp.stack([w1, w2], axis=1)
    indices = jnp.stack([i1, i2], axis=1)
    return weights.astype(x.dtype), indices


# speedup vs baseline: 2.5766x; 1.0909x over previous
import functools

import jax
import jax.numpy as jnp
from jax import lax
from jax.experimental import pallas as pl
from jax.experimental.pallas import tpu as pltpu
from jax.experimental.pallas import tpu_sc as plsc

_NT = 16384
_D = 211
_NE = 100
_EP = 104
_TBLK = 2048
_NW = 32
_CHUNK = _NT // _NW
_NG = _CHUNK // 16


_SUB = _TBLK // _CHUNK


def _score_body(xt_ref, w_ref, b_ref, o_ref):
    w = jnp.concatenate(
        [w_ref[...], jnp.zeros((_EP - _NE, _D), jnp.float32)], axis=0)
    brow = jnp.concatenate(
        [b_ref[...], jnp.zeros((1, _EP - _NE), jnp.float32)], axis=1)
    b = lax.transpose(brow, (1, 0))
    for j in range(_NW):
        logits = lax.dot_general(
            w, xt_ref[:, pl.ds(j * _CHUNK, _CHUNK)],
            dimension_numbers=(((1,), (0,)), ((), ())),
            preferred_element_type=jnp.float32,
        )
        o_ref[pl.ds(j, 1)] = jax.nn.sigmoid(logits + b)[None]


def _scores_tc(xt, w_raw, b_raw):
    return pl.pallas_call(
        _score_body,
        in_specs=[
            pl.BlockSpec(memory_space=pltpu.VMEM),
            pl.BlockSpec(memory_space=pltpu.VMEM),
            pl.BlockSpec(memory_space=pltpu.VMEM),
        ],
        out_specs=pl.BlockSpec(memory_space=pltpu.VMEM),
        out_shape=jax.ShapeDtypeStruct((_NW, _EP, _CHUNK), jnp.float32),
    )(xt, w_raw, b_raw)


def _router_sc(scores_b):
    mesh = plsc.VectorSubcoreMesh(core_axis_name="c", subcore_axis_name="s")

    @functools.partial(
        pl.kernel,
        mesh=mesh,
        out_type=[
            jax.ShapeDtypeStruct((2, _NT), jnp.float32),
            jax.ShapeDtypeStruct((2, _NT), jnp.int32),
        ],
        scratch_types=[
            pltpu.VMEM((_EP, _CHUNK), jnp.float32),
            pltpu.VMEM((_CHUNK,), jnp.float32),
            pltpu.VMEM((_CHUNK,), jnp.float32),
            pltpu.VMEM((_CHUNK,), jnp.int32),
            pltpu.VMEM((_CHUNK,), jnp.int32),
        ],
    )
    def k(scores_hbm, w_hbm, i_hbm,
          sc_v, w1_v, w2_v, i1_v, i2_v):
        wid = lax.axis_index("c") * 16 + lax.axis_index("s")
        base = wid * _CHUNK
        pltpu.sync_copy(scores_hbm.at[wid], sc_v)

        def body(g, carry):
            offs = tuple(g * 64 + 16 * t for t in range(4))

            def estep(eo, st):
                for k in range(4):
                    e = eo * 4 + k
                    col = jnp.full((16,), 1, jnp.int32) * e
                    nst = []
                    for off, (m1, m2, i1, i2) in zip(offs, st):
                        v = sc_v[pl.ds(e, 1), pl.ds(off, 16)].reshape(16)
                        gt1 = v > m1
                        gt2 = v > m2
                        m2 = jnp.where(gt1, m1, jnp.where(gt2, v, m2))
                        i2 = jnp.where(gt1, i1, jnp.where(gt2, col, i2))
                        m1 = jnp.where(gt1, v, m1)
                        i1 = jnp.where(gt1, col, i1)
                        nst.append((m1, m2, i1, i2))
                    st = tuple(nst)
                return st

            init1 = (jnp.full((16,), -jnp.inf, jnp.float32),
                     jnp.full((16,), -jnp.inf, jnp.float32),
                     jnp.zeros((16,), jnp.int32),
                     jnp.zeros((16,), jnp.int32))
            sts = lax.fori_loop(0, _NE // 4, estep, (init1,) * 4)
            for off, (m1, m2, i1, i2) in zip(offs, sts):
                s = m1 + m2
                w1_v[pl.ds(off, 16)] = m1 / s
                w2_v[pl.ds(off, 16)] = m2 / s
                i1_v[pl.ds(off, 16)] = i1
                i2_v[pl.ds(off, 16)] = i2
            return carry

        lax.fori_loop(0, _NG // 4, body, 0)
        pltpu.sync_copy(w1_v, w_hbm.at[0, pl.ds(base, _CHUNK)])
        pltpu.sync_copy(w2_v, w_hbm.at[1, pl.ds(base, _CHUNK)])
        pltpu.sync_copy(i1_v, i_hbm.at[0, pl.ds(base, _CHUNK)])
        pltpu.sync_copy(i2_v, i_hbm.at[1, pl.ds(base, _CHUNK)])

    return k(scores_b)


def kernel(x, W, b):
    scores_t = _scores_tc(x.T, W, b.reshape(1, _NE))
    w_pair, i_pair = _router_sc(scores_t)
    return w_pair.T.astype(x.dtype), i_pair.T
